# baseline (speedup 1.0000x reference)
import functools

import jax
import jax.numpy as jnp
from jax import lax
from jax.experimental import pallas as pl
from jax.experimental.pallas import tpu as pltpu
from jax.experimental.pallas import tpu_sc as plsc

ALPHA = 0.5
NC = 2
NS = 16
LANES = 16


def _deg_body(n_pad, e, chunk, row_hbm, col_hbm, w_hbm, deg_hbm,
              idx_v, w_v, deg_loc, red_v, shared):
    cid = lax.axis_index("c")
    sid = lax.axis_index("s")
    zeros = jnp.zeros((LANES,), jnp.float32)

    def zbody(i, _):
        deg_loc[pl.ds(i * LANES, LANES)] = zeros
        return 0
    lax.fori_loop(0, n_pad // LANES, zbody, 0)

    per_tile = e // NS
    base = sid * per_tile

    def chunk_body(jc, _):
        off = base + jc * chunk

        @pl.when(cid == 0)
        def _():
            pltpu.sync_copy(row_hbm.at[pl.ds(off, chunk)], idx_v)

        @pl.when(cid == 1)
        def _():
            pltpu.sync_copy(col_hbm.at[pl.ds(off, chunk)], idx_v)

        pltpu.sync_copy(w_hbm.at[pl.ds(off, chunk)], w_v)

        def inner(i, _):
            sl = pl.ds(i * LANES, LANES)
            plsc.addupdate_scatter(deg_loc, [idx_v[sl]], w_v[sl])
            return 0
        lax.fori_loop(0, chunk // LANES, inner, 0)
        return 0
    lax.fori_loop(0, per_tile // chunk, chunk_body, 0)

    pltpu.sync_copy(deg_loc, shared.at[sid])
    plsc.subcore_barrier()
    s = n_pad // NS
    cbase = sid * s
    for t in range(NS):
        pltpu.sync_copy(shared.at[t, pl.ds(cbase, s)], red_v.at[t])

    def sum_body(i, _):
        sl = pl.ds(i * LANES, LANES)
        acc = red_v[0, sl]
        for t in range(1, NS):
            acc = acc + red_v[t, sl]
        deg_loc[sl] = acc
        return 0
    lax.fori_loop(0, s // LANES, sum_body, 0)
    pltpu.sync_copy(deg_loc.at[pl.ds(0, s)], deg_hbm.at[cid, pl.ds(cbase, s)])


def _spmm_body(n_pad, e, k, r, g4d, s4d, w3d, ycat_hbm,
               p_hbm, gidx_all, sidx_all, w_all,
               *rows_acc_sems):
    rows = rows_acc_sems[:r]
    acc = rows_acc_sems[r]
    sem_g = rows_acc_sems[r + 1:2 * r + 1]
    sem_s = rows_acc_sems[2 * r + 1:3 * r + 1]

    cid = lax.axis_index("c")
    sid = lax.axis_index("s")
    zeros = jnp.zeros((LANES,), jnp.float32)
    m = gidx_all.shape[0]
    n_half = (e // NS) // k // m

    def zbody(i, _):
        for c in range(8):
            rows[0][i, pl.ds(c * LANES, LANES)] = zeros
        return 0
    lax.fori_loop(0, k, zbody, 0)
    s = n_pad // NS
    zc = 80
    for t in range(s // zc):
        pltpu.sync_copy(rows[0].at[pl.ds(0, zc)],
                        acc.at[pl.ds(sid * s + t * zc, zc)])
    plsc.subcore_barrier()

    def half_body(h, _):
        slab = sid * n_half + h

        pltpu.sync_copy(g4d.at[cid, slab], gidx_all)
        pltpu.sync_copy(s4d.at[cid, slab], sidx_all)
        pltpu.sync_copy(w3d.at[slab], w_all)

        for t in range(r - 1):
            pltpu.async_copy(ycat_hbm.at[gidx_all.at[t]], rows[t], sem_g[t])

        def ring(i, _):
            for b in range(r):
                c = i * r + b
                bm1 = (b - 1) % r
                pltpu.make_async_copy(
                    ycat_hbm.at[gidx_all.at[c]], rows[b], sem_g[b]).wait()

                @pl.when(c + r - 1 < m)
                def _(c=c, b=b, bm1=bm1):
                    @pl.when(c >= 1)
                    def _():
                        pltpu.make_async_copy(
                            rows[bm1], acc.at[sidx_all.at[c - 1]],
                            sem_s[bm1]).wait()
                    pltpu.async_copy(
                        ycat_hbm.at[gidx_all.at[c + r - 1]], rows[bm1],
                        sem_g[bm1])

                def scale(g, _, c=c, b=b):
                    w16 = w_all[c, pl.ds(g * LANES, LANES)]
                    for j in range(LANES):
                        wj = w16[j]
                        for cc in range(8):
                            sl = pl.ds(cc * LANES, LANES)
                            rows[b][g * LANES + j, sl] = (
                                rows[b][g * LANES + j, sl] * wj)
                    return 0
                lax.fori_loop(0, k // LANES, scale, 0)

                pltpu.async_copy(rows[b], acc.at[sidx_all.at[c]], sem_s[b],
                                 add=True)
            return 0
        lax.fori_loop(0, m // r, ring, 0)

        for t in range(r):
            cc = (m - r + t) % r
            pltpu.make_async_copy(rows[cc], acc.at[sidx_all.at[0]],
                                  sem_s[cc]).wait()
        return 0
    lax.fori_loop(0, n_half, half_body, 0)

    plsc.subcore_barrier()
    for t in range(s // zc):
        sl = pl.ds(sid * s + t * zc, zc)
        pltpu.sync_copy(acc.at[sl], p_hbm.at[cid, sl])


def _inv_sqrt(deg):
    return jnp.where(deg > 0, lax.rsqrt(jnp.where(deg > 0, deg, 1.0)), 0.0)


def _proj_body(nb, x_ref, w1_ref, w2_ref, do_ref, di_ref, ycat_ref):
    pid = pl.program_id(0)
    x = x_ref[...]
    dn = (((1,), (1,)), ((), ()))

    @pl.when(pid < nb)
    def _():
        ycat_ref[...] = (ALPHA * _inv_sqrt(di_ref[...])) * lax.dot_general(
            x, w1_ref[...], dn, preferred_element_type=jnp.float32)

    @pl.when(pid >= nb)
    def _():
        ycat_ref[...] = (
            (1.0 - ALPHA) * _inv_sqrt(do_ref[...])) * lax.dot_general(
            x, w2_ref[...], dn, preferred_element_type=jnp.float32)


def _combine_body(p0_ref, p1_ref, do_ref, di_ref, bmix_ref, out_ref):
    out_ref[...] = (_inv_sqrt(do_ref[...]) * p0_ref[0]
                    + _inv_sqrt(di_ref[...]) * p1_ref[0]
                    + bmix_ref[...])


@jax.jit
def kernel(x, edge_index, edge_weight, W_s2d, b_s2d, W_d2s, b_d2s):
    n, d = x.shape
    e = edge_index.shape[1]
    n_pad = ((n + 16 * NS * NC - 1) // (16 * NS * NC)) * (16 * NS * NC)

    row = edge_index[0]
    col = edge_index[1]
    x_pad = jnp.pad(x, ((0, n_pad - n), (0, 0)))

    mesh = plsc.VectorSubcoreMesh(core_axis_name="c", subcore_axis_name="s")

    chunk = 2000
    deg = pl.kernel(
        functools.partial(_deg_body, n_pad, e, chunk),
        out_type=jax.ShapeDtypeStruct((2, n_pad), jnp.float32),
        mesh=mesh,
        scratch_types=[
            pltpu.VMEM((chunk,), jnp.int32),
            pltpu.VMEM((chunk,), jnp.float32),
            pltpu.VMEM((n_pad,), jnp.float32),
            pltpu.VMEM((NS, n_pad // NS), jnp.float32),
            pltpu.VMEM_SHARED((NS, n_pad), jnp.float32),
        ],
        compiler_params=pltpu.CompilerParams(needs_layout_passes=False),
    )(row, col, edge_weight)

    do_col = deg[0].reshape(n_pad, 1)
    di_col = deg[1].reshape(n_pad, 1)
    bm = 1024
    nb = n_pad // bm
    ycat = pl.pallas_call(
        functools.partial(_proj_body, nb),
        grid=(2 * nb,),
        in_specs=[
            pl.BlockSpec((bm, d), lambda i: (i % nb, 0)),
            pl.BlockSpec((d, d), lambda i: (0, 0)),
            pl.BlockSpec((d, d), lambda i: (0, 0)),
            pl.BlockSpec((bm, 1), lambda i: (i % nb, 0)),
            pl.BlockSpec((bm, 1), lambda i: (i % nb, 0)),
        ],
        out_specs=pl.BlockSpec((bm, d), lambda i: (i, 0)),
        out_shape=jax.ShapeDtypeStruct((2 * n_pad, d), jnp.float32),
    )(x_pad, W_s2d, W_d2s, do_col, di_col)

    k = 96
    r = 3
    n_half = 9
    e_pad = ((e + NS * n_half * k - 1) // (NS * n_half * k)) * (NS * n_half * k)
    row_p = jnp.pad(row, (0, e_pad - e))
    col_p = jnp.pad(col, (0, e_pad - e))
    w_p = jnp.pad(edge_weight, (0, e_pad - e))
    m_half = (e_pad // NS) // k // n_half
    row2d = row_p.reshape(NS * n_half, m_half, k)
    col2d = col_p.reshape(NS * n_half, m_half, k)
    w3d = w_p.reshape(NS * n_half, m_half, k)
    g4d = jnp.stack([col2d, row2d + n_pad])
    s4d = jnp.stack([row2d, col2d])
    p = pl.kernel(
        functools.partial(_spmm_body, n_pad, e_pad, k, r),
        out_type=jax.ShapeDtypeStruct((2, n_pad, d), jnp.float32),
        mesh=mesh,
        scratch_types=(
            [
                pltpu.VMEM((m_half, k), jnp.int32),
                pltpu.VMEM((m_half, k), jnp.int32),
                pltpu.VMEM((m_half, k), jnp.float32),
            ]
            + [pltpu.VMEM((k, d), jnp.float32) for _ in range(r)]
            + [pltpu.VMEM_SHARED((n_pad, d), jnp.float32)]
            + [pltpu.SemaphoreType.DMA for _ in range(2 * r)]
        ),
        compiler_params=pltpu.CompilerParams(needs_layout_passes=False),
    )(g4d, s4d, w3d, ycat)

    bmix = (ALPHA * b_s2d + (1.0 - ALPHA) * b_d2s).reshape(1, d)
    bf = 2000
    out = pl.pallas_call(
        _combine_body,
        grid=(n // bf,),
        in_specs=[
            pl.BlockSpec((1, bf, d), lambda i: (0, i, 0)),
            pl.BlockSpec((1, bf, d), lambda i: (1, i, 0)),
            pl.BlockSpec((bf, 1), lambda i: (i, 0)),
            pl.BlockSpec((bf, 1), lambda i: (i, 0)),
            pl.BlockSpec((1, d), lambda i: (0, 0)),
        ],
        out_specs=pl.BlockSpec((bf, d), lambda i: (i, 0)),
        out_shape=jax.ShapeDtypeStruct((n, d), jnp.float32),
    )(p, p, do_col, di_col, bmix)
    return out


# speedup vs baseline: 2.7974x; 2.7974x over previous
import functools

import jax
import jax.numpy as jnp
from jax import lax
from jax.experimental import pallas as pl
from jax.experimental.pallas import tpu as pltpu
from jax.experimental.pallas import tpu_sc as plsc

ALPHA = 0.5
NC = 2
NS = 16
LANES = 16


def _deg_body(n_pad, e, chunk, row_hbm, col_hbm, w_hbm, deg_hbm,
              idx_v, w_v, deg_loc, red_v, shared):
    cid = lax.axis_index("c")
    sid = lax.axis_index("s")
    zeros = jnp.zeros((LANES,), jnp.float32)

    def zbody(i, _):
        deg_loc[pl.ds(i * LANES, LANES)] = zeros
        return 0
    lax.fori_loop(0, n_pad // LANES, zbody, 0)

    per_tile = e // NS
    base = sid * per_tile

    def chunk_body(jc, _):
        off = base + jc * chunk

        @pl.when(cid == 0)
        def _():
            pltpu.sync_copy(row_hbm.at[pl.ds(off, chunk)], idx_v)

        @pl.when(cid == 1)
        def _():
            pltpu.sync_copy(col_hbm.at[pl.ds(off, chunk)], idx_v)

        pltpu.sync_copy(w_hbm.at[pl.ds(off, chunk)], w_v)

        def inner(i, _):
            sl = pl.ds(i * LANES, LANES)
            plsc.addupdate_scatter(deg_loc, [idx_v[sl]], w_v[sl])
            return 0
        lax.fori_loop(0, chunk // LANES, inner, 0)
        return 0
    lax.fori_loop(0, per_tile // chunk, chunk_body, 0)

    pltpu.sync_copy(deg_loc, shared.at[sid])
    plsc.subcore_barrier()
    s = n_pad // NS
    cbase = sid * s
    for t in range(NS):
        pltpu.sync_copy(shared.at[t, pl.ds(cbase, s)], red_v.at[t])

    def sum_body(i, _):
        sl = pl.ds(i * LANES, LANES)
        acc = red_v[0, sl]
        for t in range(1, NS):
            acc = acc + red_v[t, sl]
        deg_loc[sl] = acc
        return 0
    lax.fori_loop(0, s // LANES, sum_body, 0)
    pltpu.sync_copy(deg_loc.at[pl.ds(0, s)], deg_hbm.at[cid, pl.ds(cbase, s)])


def _spmm_body(n_pad, e, k, r, g4d, s4d, w3d, ycat_hbm,
               p_hbm, gidx_all, sidx_all, w_all,
               *rows_acc_sems):
    rows = rows_acc_sems[:r]
    acc = rows_acc_sems[r]
    sem_g = rows_acc_sems[r + 1:2 * r + 1]
    sem_s = rows_acc_sems[2 * r + 1:3 * r + 1]

    cid = lax.axis_index("c")
    sid = lax.axis_index("s")
    zeros = jnp.zeros((LANES,), jnp.float32)
    m = gidx_all.shape[0]
    n_half = (e // NS) // k // m

    def zbody(i, _):
        for c in range(8):
            rows[0][i, pl.ds(c * LANES, LANES)] = zeros
        return 0
    lax.fori_loop(0, k, zbody, 0)
    s = n_pad // NS
    zc = 80
    for t in range(s // zc):
        pltpu.sync_copy(rows[0].at[pl.ds(0, zc)],
                        acc.at[pl.ds(sid * s + t * zc, zc)])
    plsc.subcore_barrier()

    def half_body(h, _):
        slab = sid * n_half + h

        pltpu.sync_copy(g4d.at[cid, slab], gidx_all)
        pltpu.sync_copy(s4d.at[cid, slab], sidx_all)
        pltpu.sync_copy(w3d.at[slab], w_all)

        for t in range(r - 1):
            pltpu.async_copy(ycat_hbm.at[gidx_all.at[t]], rows[t], sem_g[t])

        def ring(i, _):
            for b in range(r):
                c = i * r + b
                bm1 = (b - 1) % r
                pltpu.make_async_copy(
                    ycat_hbm.at[gidx_all.at[c]], rows[b], sem_g[b]).wait()

                @pl.when(c + r - 1 < m)
                def _(c=c, b=b, bm1=bm1):
                    @pl.when(c >= 1)
                    def _():
                        pltpu.make_async_copy(
                            rows[bm1], acc.at[sidx_all.at[c - 1]],
                            sem_s[bm1]).wait()
                    pltpu.async_copy(
                        ycat_hbm.at[gidx_all.at[c + r - 1]], rows[bm1],
                        sem_g[bm1])

                def scale(g, _, c=c, b=b):
                    w16 = w_all[c, pl.ds(g * LANES, LANES)]
                    for j in range(LANES):
                        wj = w16[j]
                        for cc in range(8):
                            sl = pl.ds(cc * LANES, LANES)
                            rows[b][g * LANES + j, sl] = (
                                rows[b][g * LANES + j, sl] * wj)
                    return 0
                lax.fori_loop(0, k // LANES, scale, 0)

                pltpu.async_copy(rows[b], acc.at[sidx_all.at[c]], sem_s[b],
                                 add=True)
            return 0
        lax.fori_loop(0, m // r, ring, 0)

        for t in range(r):
            cc = (m - r + t) % r
            pltpu.make_async_copy(rows[cc], acc.at[sidx_all.at[0]],
                                  sem_s[cc]).wait()
        return 0
    lax.fori_loop(0, n_half, half_body, 0)

    plsc.subcore_barrier()
    for t in range(s // zc):
        sl = pl.ds(sid * s + t * zc, zc)
        pltpu.sync_copy(acc.at[sl], p_hbm.at[cid, sl])


def _inv_sqrt(deg):
    return jnp.where(deg > 0, lax.rsqrt(jnp.where(deg > 0, deg, 1.0)), 0.0)


def _proj_body(nb, x_ref, w1_ref, w2_ref, do_ref, di_ref, ycat_ref):
    pid = pl.program_id(0)
    x = x_ref[...]
    dn = (((1,), (1,)), ((), ()))

    @pl.when(pid < nb)
    def _():
        ycat_ref[...] = (ALPHA * _inv_sqrt(di_ref[...])) * lax.dot_general(
            x, w1_ref[...], dn, preferred_element_type=jnp.float32)

    @pl.when(pid >= nb)
    def _():
        ycat_ref[...] = (
            (1.0 - ALPHA) * _inv_sqrt(do_ref[...])) * lax.dot_general(
            x, w2_ref[...], dn, preferred_element_type=jnp.float32)


def _combine_body(p0_ref, p1_ref, do_ref, di_ref, bmix_ref, out_ref):
    out_ref[...] = (_inv_sqrt(do_ref[...]) * p0_ref[0]
                    + _inv_sqrt(di_ref[...]) * p1_ref[0]
                    + bmix_ref[...])


@jax.jit
def kernel(x, edge_index, edge_weight, W_s2d, b_s2d, W_d2s, b_d2s):
    n, d = x.shape
    e = edge_index.shape[1]
    n_pad = ((n + 16 * NS * NC - 1) // (16 * NS * NC)) * (16 * NS * NC)

    row = edge_index[0]
    col = edge_index[1]
    x_pad = jnp.pad(x, ((0, n_pad - n), (0, 0)))

    mesh = plsc.VectorSubcoreMesh(core_axis_name="c", subcore_axis_name="s")

    chunk = 2000
    deg = pl.kernel(
        functools.partial(_deg_body, n_pad, e, chunk),
        out_type=jax.ShapeDtypeStruct((2, n_pad), jnp.float32),
        mesh=mesh,
        scratch_types=[
            pltpu.VMEM((chunk,), jnp.int32),
            pltpu.VMEM((chunk,), jnp.float32),
            pltpu.VMEM((n_pad,), jnp.float32),
            pltpu.VMEM((NS, n_pad // NS), jnp.float32),
            pltpu.VMEM_SHARED((NS, n_pad), jnp.float32),
        ],
        compiler_params=pltpu.CompilerParams(needs_layout_passes=False),
    )(row, col, edge_weight)

    do_col = deg[0].reshape(n_pad, 1)
    di_col = deg[1].reshape(n_pad, 1)
    bm = 1024
    nb = n_pad // bm
    ycat = pl.pallas_call(
        functools.partial(_proj_body, nb),
        grid=(2 * nb,),
        in_specs=[
            pl.BlockSpec((bm, d), lambda i: (i % nb, 0)),
            pl.BlockSpec((d, d), lambda i: (0, 0)),
            pl.BlockSpec((d, d), lambda i: (0, 0)),
            pl.BlockSpec((bm, 1), lambda i: (i % nb, 0)),
            pl.BlockSpec((bm, 1), lambda i: (i % nb, 0)),
        ],
        out_specs=pl.BlockSpec((bm, d), lambda i: (i, 0)),
        out_shape=jax.ShapeDtypeStruct((2 * n_pad, d), jnp.float32),
    )(x_pad, W_s2d, W_d2s, do_col, di_col)

    k = 96
    r = 3
    n_half = 9
    e_pad = ((e + NS * n_half * k - 1) // (NS * n_half * k)) * (NS * n_half * k)
    dummy = jnp.arange(e_pad - e, dtype=jnp.int32) % n_pad
    row_p = jnp.concatenate([row, dummy])
    col_p = jnp.concatenate([col, dummy])
    w_p = jnp.pad(edge_weight, (0, e_pad - e))
    m_half = (e_pad // NS) // k // n_half
    row2d = row_p.reshape(NS * n_half, m_half, k)
    col2d = col_p.reshape(NS * n_half, m_half, k)
    w3d = w_p.reshape(NS * n_half, m_half, k)
    g4d = jnp.stack([col2d, row2d + n_pad])
    s4d = jnp.stack([row2d, col2d])
    p = pl.kernel(
        functools.partial(_spmm_body, n_pad, e_pad, k, r),
        out_type=jax.ShapeDtypeStruct((2, n_pad, d), jnp.float32),
        mesh=mesh,
        scratch_types=(
            [
                pltpu.VMEM((m_half, k), jnp.int32),
                pltpu.VMEM((m_half, k), jnp.int32),
                pltpu.VMEM((m_half, k), jnp.float32),
            ]
            + [pltpu.VMEM((k, d), jnp.float32) for _ in range(r)]
            + [pltpu.VMEM_SHARED((n_pad, d), jnp.float32)]
            + [pltpu.SemaphoreType.DMA for _ in range(2 * r)]
        ),
        compiler_params=pltpu.CompilerParams(needs_layout_passes=False),
    )(g4d, s4d, w3d, ycat)

    bmix = (ALPHA * b_s2d + (1.0 - ALPHA) * b_d2s).reshape(1, d)
    bf = 2000
    out = pl.pallas_call(
        _combine_body,
        grid=(n // bf,),
        in_specs=[
            pl.BlockSpec((1, bf, d), lambda i: (0, i, 0)),
            pl.BlockSpec((1, bf, d), lambda i: (1, i, 0)),
            pl.BlockSpec((bf, 1), lambda i: (i, 0)),
            pl.BlockSpec((bf, 1), lambda i: (i, 0)),
            pl.BlockSpec((1, d), lambda i: (0, 0)),
        ],
        out_specs=pl.BlockSpec((bf, d), lambda i: (i, 0)),
        out_shape=jax.ShapeDtypeStruct((n, d), jnp.float32),
    )(p, p, do_col, di_col, bmix)
    return out
